# Initial kernel scaffold; baseline (speedup 1.0000x reference)
import jax
import jax.numpy as jnp
from jax.experimental import pallas as pl


def kernel(output, target, ind, mask, cat):
    raise NotImplementedError("write your pallas kernel here")


# speedup vs baseline: 2.4025x; 2.4025x over previous
import functools

import jax
import jax.numpy as jnp
from jax import lax
from jax.experimental import pallas as pl
from jax.experimental.pallas import tpu as pltpu
from jax.experimental.pallas import tpu_sc as plsc

B, C, H, W, M = 64, 8, 128, 128, 128
HW = H * W
CHW = C * HW
N = B * M
NW = 32
EPW = N // NW
VPW = EPW // 16


def _make_mesh():
    return plsc.VectorSubcoreMesh(core_axis_name="c", subcore_axis_name="s")


@functools.partial(
    pl.kernel,
    mesh=_make_mesh(),
    out_type=[
        jax.ShapeDtypeStruct((NW, 16), jnp.float32),
        jax.ShapeDtypeStruct((NW, 16), jnp.float32),
    ],
    scratch_types=[
        pltpu.VMEM((EPW,), jnp.int32),
        pltpu.VMEM((EPW,), jnp.int32),
        pltpu.VMEM((EPW,), jnp.float32),
        pltpu.VMEM((EPW,), jnp.float32),
        pltpu.VMEM((2, 128), jnp.int32),
        pltpu.VMEM((2, 128), jnp.float32),
        pltpu.VMEM((16,), jnp.float32),
        pltpu.VMEM((16,), jnp.float32),
        pltpu.SemaphoreType.DMA,
    ],
)
def _gather_partials(feat_hbm, ind_hbm, cat_hbm, tgt_hbm, msk_hbm,
                     loss_out, msk_out,
                     ind_v, cat_v, tgt_v, msk_v, gidx_v, vals_v,
                     lstage, mstage, sem):
    wid = lax.axis_index("s") * 2 + lax.axis_index("c")
    base = wid * EPW
    pltpu.sync_copy(ind_hbm.at[pl.ds(base, EPW)], ind_v)
    pltpu.sync_copy(cat_hbm.at[pl.ds(base, EPW)], cat_v)
    pltpu.sync_copy(tgt_hbm.at[pl.ds(base, EPW)], tgt_v)
    pltpu.sync_copy(msk_hbm.at[pl.ds(base, EPW)], msk_v)
    b0 = wid * 2
    for i in range(VPW):
        g = (ind_v[pl.ds(i * 16, 16)]
             + cat_v[pl.ds(i * 16, 16)] * HW
             + (b0 + (i // 8)) * CHW)
        gidx_v[i // 8, pl.ds((i % 8) * 16, 16)] = g
    copies = [
        pltpu.async_copy(feat_hbm.at[gidx_v.at[j]], vals_v.at[j], sem)
        for j in range(2)
    ]
    for cp in copies:
        cp.wait()
    acc = jnp.zeros((16,), jnp.float32)
    mac = jnp.zeros((16,), jnp.float32)
    for i in range(VPW):
        v = vals_v[i // 8, pl.ds((i % 8) * 16, 16)]
        m = msk_v[pl.ds(i * 16, 16)]
        t = tgt_v[pl.ds(i * 16, 16)]
        acc = acc + jnp.abs(v * m - t * m)
        mac = mac + m
    lstage[...] = acc
    mstage[...] = mac
    pltpu.sync_copy(lstage, loss_out.at[wid])
    pltpu.sync_copy(mstage, msk_out.at[wid])


@functools.partial(
    pl.kernel,
    mesh=_make_mesh(),
    out_type=jax.ShapeDtypeStruct((16,), jnp.float32),
    scratch_types=[
        pltpu.VMEM((NW, 16), jnp.float32),
        pltpu.VMEM((NW, 16), jnp.float32),
        pltpu.VMEM((16,), jnp.float32),
        pltpu.VMEM((16,), jnp.float32),
    ],
)
def _finalize(loss_hbm, msk_hbm, out_hbm, loss_v, msk_v, stage, stage2):
    wid = lax.axis_index("s") * 2 + lax.axis_index("c")

    @pl.when(wid == 0)
    def _():
        pltpu.sync_copy(loss_hbm, loss_v)
        pltpu.sync_copy(msk_hbm, msk_v)
        acc = jnp.zeros((16,), jnp.float32)
        mac = jnp.zeros((16,), jnp.float32)
        for i in range(NW):
            acc = acc + loss_v[i]
            mac = mac + msk_v[i]
        lanes = lax.iota(jnp.int32, 16)
        dnums = lax.GatherDimensionNumbers(
            offset_dims=(), collapsed_slice_dims=(0,), start_index_map=(0,))
        shuffle = lambda x, perm: lax.gather(
            x, perm[:, None], dnums, slice_sizes=(1,),
            mode=lax.GatherScatterMode.PROMISE_IN_BOUNDS)
        for k in (1, 2, 4, 8):
            perm = lanes ^ k
            acc = acc + shuffle(acc, perm)
            mac = mac + shuffle(mac, perm)
        res = acc / (mac + 1e-6)
        stage[...] = res
        pltpu.sync_copy(stage, out_hbm)


def kernel(output, target, ind, mask, cat):
    feat = output.reshape(-1)
    ind32 = ind.astype(jnp.int32).reshape(-1)
    cat32 = cat.astype(jnp.int32).reshape(-1)
    tgt = target.reshape(-1)
    msk = mask.reshape(-1)
    loss_p, msk_p = _gather_partials(feat, ind32, cat32, tgt, msk)
    res = _finalize(loss_p, msk_p)
    return res[0]


# speedup vs baseline: 3.0246x; 1.2589x over previous
import functools

import jax
import jax.numpy as jnp
from jax import lax
from jax.experimental import pallas as pl
from jax.experimental.pallas import tpu as pltpu
from jax.experimental.pallas import tpu_sc as plsc

B, C, H, W, M = 64, 8, 128, 128, 128
HW = H * W
CHW = C * HW
N = B * M
NT = 16
EPW = N // NT
VPW = EPW // 16
NCH = EPW // 128


@functools.partial(
    pl.kernel,
    mesh=plsc.VectorSubcoreMesh(core_axis_name="c", subcore_axis_name="s"),
    out_type=[jax.ShapeDtypeStruct((16,), jnp.float32),
              jax.ShapeDtypeStruct((NT, 2, 16), jnp.float32)],
    scratch_types=[
        pltpu.VMEM((EPW,), jnp.int32),
        pltpu.VMEM((EPW,), jnp.int32),
        pltpu.VMEM((EPW,), jnp.float32),
        pltpu.VMEM((EPW,), jnp.float32),
        pltpu.VMEM((VPW, 16), jnp.float32),
        pltpu.VMEM((2, 16), jnp.float32),
        pltpu.VMEM((NT, 2, 16), jnp.float32),
        pltpu.VMEM((16,), jnp.float32),
        pltpu.SemaphoreType.DMA,
        pltpu.SemaphoreType.DMA,
        pltpu.SemaphoreType.DMA,
    ],
)
def _depth_loss(feat_hbm, ind_hbm, cat_hbm, tgt_hbm, msk_hbm,
                out_hbm, parts_hbm,
                ind_v, cat_v, tgt_v, msk_v, vals_v,
                part_v, allp_v, stage, sem_i, sem_f, sem_g):
    cid = lax.axis_index("c")
    sid = lax.axis_index("s")

    @pl.when(cid == 0)
    def _():
        base = sid * EPW
        cp_ind = pltpu.async_copy(ind_hbm.at[pl.ds(base, EPW)], ind_v, sem_i)
        cp_cat = pltpu.async_copy(cat_hbm.at[pl.ds(base, EPW)], cat_v, sem_i)
        cp_tgt = pltpu.async_copy(tgt_hbm.at[pl.ds(base, EPW)], tgt_v, sem_f)
        cp_msk = pltpu.async_copy(msk_hbm.at[pl.ds(base, EPW)], msk_v, sem_f)
        cp_ind.wait()
        cp_cat.wait()
        b0 = sid * (EPW // M)
        gathers = []
        for v in range(VPW):
            g = (ind_v[pl.ds(v * 16, 16)]
                 + cat_v[pl.ds(v * 16, 16)] * HW
                 + (b0 + v // 8) * CHW)
            gathers.append(
                pltpu.async_copy(feat_hbm.at[g], vals_v.at[v], sem_g))
        cp_tgt.wait()
        cp_msk.wait()
        for cp in gathers:
            cp.wait()
        acc = jnp.zeros((16,), jnp.float32)
        mac = jnp.zeros((16,), jnp.float32)
        for v in range(VPW):
            val = vals_v[v, :]
            m = msk_v[pl.ds(v * 16, 16)]
            t = tgt_v[pl.ds(v * 16, 16)]
            acc = acc + jnp.abs(val * m - t * m)
            mac = mac + m
        part_v[0, :] = acc
        part_v[1, :] = mac
        pltpu.sync_copy(part_v, parts_hbm.at[sid])
        plsc.subcore_barrier()

        @pl.when(sid == 0)
        def _():
            pltpu.sync_copy(parts_hbm, allp_v)
            facc = jnp.zeros((16,), jnp.float32)
            fmac = jnp.zeros((16,), jnp.float32)
            for i in range(NT):
                facc = facc + allp_v[i, 0, :]
                fmac = fmac + allp_v[i, 1, :]
            lanes = lax.iota(jnp.int32, 16)
            dnums = lax.GatherDimensionNumbers(
                offset_dims=(), collapsed_slice_dims=(0,),
                start_index_map=(0,))
            shuffle = lambda x, perm: lax.gather(
                x, perm[:, None], dnums, slice_sizes=(1,),
                mode=lax.GatherScatterMode.PROMISE_IN_BOUNDS)
            for k in (1, 2, 4, 8):
                perm = lanes ^ k
                facc = facc + shuffle(facc, perm)
                fmac = fmac + shuffle(fmac, perm)
            res = facc / (fmac + 1e-6)
            stage[...] = res
            pltpu.sync_copy(stage, out_hbm)


def kernel(output, target, ind, mask, cat):
    feat = output.reshape(-1)
    ind32 = ind.astype(jnp.int32).reshape(-1)
    cat32 = cat.astype(jnp.int32).reshape(-1)
    tgt = target.reshape(-1)
    msk = mask.reshape(-1)
    res, _ = _depth_loss(feat, ind32, cat32, tgt, msk)
    return res[0]
